# baseline (speedup 1.0000x reference)
import functools

import jax
import jax.numpy as jnp
from jax import lax
from jax.experimental import pallas as pl
from jax.experimental.pallas import tpu as pltpu
from jax.experimental.pallas import tpu_sc as plsc

NEG = -1e9
_PN = 384 * 384
_HALF = _PN // 2
_NCH = _HALF // 16

_SC_MESH = plsc.VectorSubcoreMesh(core_axis_name="c", subcore_axis_name="s")
_SC_PARAMS = pltpu.CompilerParams(needs_layout_passes=False)


def _enc_kernel(gh_ref, gah_ref, text_ref, link_ref, conf_ref,
                enc_ref, stats_ref):
    conf = conf_ref[...]
    gh = gh_ref[...]
    gah = gah_ref[...]
    loss_g = (text_ref[...] - gh) ** 2 * conf
    loss_a = (link_ref[...] - gah) ** 2 * conf
    pos_g = gh >= 0.1
    pos_a = gah >= 0.1

    def rs(x):
        return jnp.sum(x, axis=(1, 2), keepdims=True).reshape(8, 1)

    enc_g = lax.bitcast_convert_type(jnp.where(pos_g, -loss_g, loss_g), jnp.int32)
    enc_a = lax.bitcast_convert_type(jnp.where(pos_a, -loss_a, loss_a), jnp.int32)
    enc_ref[...] = jnp.concatenate([enc_g, enc_a], axis=0)

    pc = jnp.concatenate([rs(jnp.where(pos_g, 1.0, 0.0)),
                          rs(jnp.where(pos_a, 1.0, 0.0))], axis=0)
    sp = jnp.concatenate([rs(jnp.where(pos_g, loss_g, 0.0)),
                          rs(jnp.where(pos_a, loss_a, 0.0))], axis=0)
    sn = jnp.concatenate([rs(jnp.where(pos_g, 0.0, loss_g)),
                          rs(jnp.where(pos_a, 0.0, loss_a))], axis=0)
    lane = lax.broadcasted_iota(jnp.int32, (16, 128), 1)
    stats_ref[...] = (jnp.where(lane == 0, 3.0 * pc, 0.0)
                      + jnp.where(lane == 1, pc, 0.0)
                      + jnp.where(lane == 2, sp, 0.0)
                      + jnp.where(lane == 3, sn, 0.0))


def _sc_hist_sweep(data_v, level, b_n, b_a, cn_v, ca_v, sn_v, sa_v):
    ones = jnp.ones((16,), jnp.int32)

    def body(i, _):
        v = data_v[pl.ds(i * 16, 16)]
        va = jnp.bitwise_and(v, 0x7FFFFFFF)
        if level == 0:
            mn = v >= 0
            ma = va >= 0
            idxn = jnp.bitwise_and(lax.shift_right_logical(v, 20), 0x7FF)
            idxa = jnp.bitwise_and(lax.shift_right_logical(va, 20), 0x7FF)
        elif level == 1:
            mn = jnp.logical_and(v >= 0, lax.shift_right_logical(v, 20) == b_n)
            ma = lax.shift_right_logical(va, 20) == b_a
            idxn = jnp.bitwise_and(lax.shift_right_logical(v, 9), 0x7FF)
            idxa = jnp.bitwise_and(lax.shift_right_logical(va, 9), 0x7FF)
        else:
            mn = jnp.logical_and(v >= 0, lax.shift_right_logical(v, 9) == b_n)
            ma = lax.shift_right_logical(va, 9) == b_a
            idxn = jnp.bitwise_and(v, 0x1FF)
            idxa = jnp.bitwise_and(va, 0x1FF)
        fn = plsc.bitcast(v, jnp.float32)
        fa = plsc.bitcast(va, jnp.float32)
        plsc.addupdate_scatter(cn_v, [idxn], ones, mask=mn)
        plsc.addupdate_scatter(ca_v, [idxa], ones, mask=ma)
        plsc.addupdate_scatter(sn_v, [idxn], fn, mask=mn)
        plsc.addupdate_scatter(sa_v, [idxa], fa, mask=ma)
        return 0

    lax.fori_loop(0, _NCH, body, 0, unroll=False)


def _sc_scan_hist(cnt_v, sum_v, nbuck, k):
    nch = nbuck // 16

    def tbody(j, carry):
        t, ts = carry
        return (t + jnp.sum(cnt_v[pl.ds(j * 16, 16)]),
                ts + jnp.sum(sum_v[pl.ds(j * 16, 16)]))

    total, totals = lax.fori_loop(
        0, nch, tbody, (jnp.int32(0), jnp.float32(0.0)), unroll=False)

    def sbody(j, carry):
        acc, bcnt, cle, sle = carry
        h = cnt_v[pl.ds(j * 16, 16)]
        hs = sum_v[pl.ds(j * 16, 16)]
        csum = plsc.cumsum(h)
        pref_excl = (acc + csum) - h
        suffix = total - pref_excl
        m = suffix >= k
        bcnt = bcnt + jnp.sum(jnp.where(m, 1, 0))
        cle = cle + jnp.sum(jnp.where(m, h, 0))
        sle = sle + jnp.sum(jnp.where(m, hs, 0.0))
        return (acc + jnp.sum(h), bcnt, cle, sle)

    _, bcnt, cle, sle = lax.fori_loop(
        0, nch, sbody, (jnp.int32(0), jnp.int32(0), jnp.int32(0),
                        jnp.float32(0.0)), unroll=False)
    b = bcnt - 1
    return b, total - cle, totals - sle


def _sc_zero(ref, n):
    z = jnp.zeros((16,), jnp.int32)

    def body(i, _):
        ref[pl.ds(i * 16, 16)] = z
        return 0
    lax.fori_loop(0, n // 16, body, 0, unroll=False)


def _sc_zero_f(ref, n):
    z = jnp.zeros((16,), jnp.float32)

    def body(i, _):
        ref[pl.ds(i * 16, 16)] = z
        return 0
    lax.fori_loop(0, n // 16, body, 0, unroll=False)


def _sc_exchange(local_v, partner_v, shared, s, parity, n, sem_unused=None):
    pltpu.sync_copy(local_v.at[pl.ds(0, n)], shared.at[s, parity, pl.ds(0, n)])
    plsc.subcore_barrier()
    sp = jnp.bitwise_xor(s, 1)
    pltpu.sync_copy(shared.at[sp, parity, pl.ds(0, n)], partner_v.at[pl.ds(0, n)])

    def body(i, _):
        local_v[pl.ds(i * 16, 16)] = local_v[pl.ds(i * 16, 16)] + partner_v[pl.ds(i * 16, 16)]
        return 0
    lax.fori_loop(0, n // 16, body, 0, unroll=False)


def _select_kernel(enc_hbm, stats_hbm, out_hbm,
                   data_v, cn_v, ca_v, pn_v, pa_v,
                   sn_v, sa_v, qn_v, qa_v,
                   param_v, res_v, shc, shs):
    c = lax.axis_index("c")
    s = lax.axis_index("s")
    row = c * 8 + lax.shift_right_logical(s, 1)
    half = jnp.bitwise_and(s, 1)

    pltpu.sync_copy(enc_hbm.at[row, pl.ds(half * _HALF, _HALF)], data_v)
    pltpu.sync_copy(stats_hbm.at[row, pl.ds(0, 16)], param_v)
    pv = param_v[...]
    li = lax.broadcasted_iota(jnp.int32, (16,), 0)
    k3f = jnp.sum(jnp.where(li == 0, pv, 0.0))
    k_n = k3f.astype(jnp.int32)
    k_a = jnp.int32(500)

    b_n = jnp.int32(0)
    b_a = jnp.int32(0)
    cgt_n = jnp.int32(0)
    cgt_a = jnp.int32(0)
    sgt_n = jnp.float32(0.0)
    sgt_a = jnp.float32(0.0)

    for level, nbuck in ((0, 2048), (1, 2048), (2, 512)):
        _sc_zero(cn_v, nbuck)
        _sc_zero(ca_v, nbuck)
        _sc_zero_f(sn_v, nbuck)
        _sc_zero_f(sa_v, nbuck)
        _sc_hist_sweep(data_v, level, b_n, b_a, cn_v, ca_v, sn_v, sa_v)
        _sc_exchange(cn_v, pn_v, shc, s, 2 * level, nbuck)
        _sc_exchange(ca_v, pa_v, shc, s, 2 * level + 1, nbuck)
        _sc_exchange(sn_v, qn_v, shs, s, 2 * level, nbuck)
        _sc_exchange(sa_v, qa_v, shs, s, 2 * level + 1, nbuck)
        bn, cn, sn = _sc_scan_hist(cn_v, sn_v, nbuck, k_n - cgt_n)
        ba, ca_, sa_ = _sc_scan_hist(ca_v, sa_v, nbuck, k_a - cgt_a)
        cgt_n = cgt_n + cn
        cgt_a = cgt_a + ca_
        sgt_n = sgt_n + sn
        sgt_a = sgt_a + sa_
        if level == 0:
            b_n, b_a = bn, ba
        elif level == 1:
            b_n = b_n * 2048 + bn
            b_a = b_a * 2048 + ba
        else:
            b_n = b_n * 512 + bn
            b_a = b_a * 512 + ba
        plsc.subcore_barrier()

    tvn = plsc.bitcast(jnp.full((16,), 1, jnp.int32) * b_n, jnp.float32)
    tva = plsc.bitcast(jnp.full((16,), 1, jnp.int32) * b_a, jnp.float32)
    remn = (k_n - cgt_n).astype(jnp.float32)
    rema = (k_a - cgt_a).astype(jnp.float32)
    tk_n = sgt_n + remn * tvn
    tk_a = sgt_a + rema * tva
    res = (jnp.where(li == 0, tk_n, 0.0) + jnp.where(li == 1, tk_a, 0.0))
    res_v[...] = res

    @pl.when(half == 0)
    def _():
        pltpu.sync_copy(res_v, out_hbm.at[row])


@functools.partial(
    pl.kernel, mesh=_SC_MESH, compiler_params=_SC_PARAMS,
    out_type=jax.ShapeDtypeStruct((16, 16), jnp.float32),
    scratch_types=[
        pltpu.VMEM((_HALF,), jnp.int32),
        pltpu.VMEM((2048,), jnp.int32), pltpu.VMEM((2048,), jnp.int32),
        pltpu.VMEM((2048,), jnp.int32), pltpu.VMEM((2048,), jnp.int32),
        pltpu.VMEM((2048,), jnp.float32), pltpu.VMEM((2048,), jnp.float32),
        pltpu.VMEM((2048,), jnp.float32), pltpu.VMEM((2048,), jnp.float32),
        pltpu.VMEM((16,), jnp.float32), pltpu.VMEM((16,), jnp.float32),
        pltpu.VMEM_SHARED((16, 6, 2048), jnp.int32),
        pltpu.VMEM_SHARED((16, 6, 2048), jnp.float32),
    ],
)
def _sc_select(enc_hbm, stats_hbm, out_hbm, *scratch):
    _select_kernel(enc_hbm, stats_hbm, out_hbm, *scratch)


def _rest_kernel(stats_ref, sc_ref, a_log_ref, p_log_ref, a_lab_ref,
                 p_lab_ref, lpt_ref, ext_ref, skip_ref, tlen_ref,
                 out_ref, lpe_ref):
    stats = stats_ref[...]
    scres = sc_ref[...]
    lane128 = lax.broadcasted_iota(jnp.int32, (16, 128), 1)
    lane16 = lax.broadcasted_iota(jnp.int32, (16, 16), 1)

    def pick(x, lanes, j):
        return jnp.sum(jnp.where(lanes == j, x, 0.0), axis=1, keepdims=True)

    k3 = pick(stats, lane128, 0)
    pcnt = pick(stats, lane128, 1)
    spos = pick(stats, lane128, 2)
    sneg = pick(stats, lane128, 3)
    tk = pick(scres, lane16, 0)
    t500 = pick(scres, lane16, 1)
    ncnt = float(_PN) - pcnt

    posi = spos / jnp.maximum(pcnt, 1.0)
    mean_neg = sneg / jnp.maximum(ncnt, 1.0)
    topk_neg = tk / jnp.maximum(k3, 1.0)
    nega = jnp.where(ncnt < k3, mean_neg, topk_neg)
    contrib = jnp.where(pcnt > 0, posi + nega, t500 / 500.0)
    saliency = jnp.sum(contrib) / 8.0

    def ce(logits, labels2d):
        n, cdim = logits.shape
        m = jnp.max(logits, axis=1, keepdims=True)
        ls = logits - m - jnp.log(jnp.sum(jnp.exp(logits - m), axis=1, keepdims=True))
        oh = lax.broadcasted_iota(jnp.int32, (n, cdim), 1) == labels2d
        return -jnp.sum(jnp.where(oh, ls, 0.0)) / float(n)

    direction = 0.5 * ce(p_log_ref[...], p_lab_ref[...]) + \
        0.5 * ce(a_log_ref[...], a_lab_ref[...])

    lpt = lpt_ref[...]
    ext = ext_ref[...]
    skipf = skip_ref[...]
    tlen = tlen_ref[...]
    N, T, C = lpt.shape
    L = ext.shape[1]

    oh = (ext[:, :, None] == lax.broadcasted_iota(jnp.int32, (N, L, C), 2))
    oh = oh.astype(jnp.float32)
    for n_i in range(N):
        lpe_ref[:, n_i, :] = lax.dot_general(
            lpt[n_i], oh[n_i], (((1,), (1,)), ((), ())),
            precision=lax.Precision.HIGHEST)

    li = lax.broadcasted_iota(jnp.int32, (N, L), 1)
    alpha0 = jnp.where(li <= 1, lpe_ref[0], NEG)

    def ctc_step(t, alpha):
        lp_t = lpe_ref[pl.ds(t, 1)].reshape(N, L)
        a1 = jnp.where(li >= 1, pltpu.roll(alpha, 1, 1), NEG)
        a2 = jnp.where(li >= 2, pltpu.roll(alpha, 2, 1), NEG)
        a2 = jnp.where(skipf > 0, a2, NEG)
        m = jnp.maximum(jnp.maximum(alpha, a1), a2)
        new = m + jnp.log(jnp.exp(alpha - m) + jnp.exp(a1 - m) + jnp.exp(a2 - m))
        new = new + lp_t
        return jnp.maximum(new, NEG)

    alpha = lax.fori_loop(1, T, ctc_step, alpha0)

    tl_i = tlen.astype(jnp.int32)
    i1 = jnp.clip(2 * tl_i, 0, L - 1)
    i2 = jnp.clip(2 * tl_i - 1, 0, L - 1)
    v1 = jnp.sum(jnp.where(li == i1, alpha, 0.0), axis=1, keepdims=True)
    v2 = jnp.sum(jnp.where(li == i2, alpha, 0.0), axis=1, keepdims=True)
    m = jnp.maximum(v1, v2)
    ll = m + jnp.log(jnp.exp(v1 - m) + jnp.exp(v2 - m))
    closs = -ll
    closs = jnp.where(closs < 1e8, closs, 0.0)
    recognition = 10.0 * jnp.mean(closs / jnp.maximum(tlen, 1.0))

    total = saliency + recognition
    lane = lax.broadcasted_iota(jnp.int32, (8, 128), 1)
    out = (jnp.where(lane == 0, total, 0.0) + jnp.where(lane == 1, saliency, 0.0)
           + jnp.where(lane == 2, direction, 0.0)
           + jnp.where(lane == 3, recognition, 0.0))
    out_ref[...] = out


@jax.jit
def _run(gh_label, gah_label, text_map, link_map, conf_map, a_logits, p_logits,
         a_label, p_label, log_probs, targets, target_lengths):
    N, S = targets.shape
    L = 2 * S + 1
    ext = jnp.zeros((N, L), dtype=targets.dtype)
    ext = ext.at[:, 1::2].set(targets)
    prev2 = jnp.concatenate(
        [jnp.full((N, 2), -1, dtype=ext.dtype), ext[:, :-2]], axis=1)
    allow_skip = ((ext != 0) & (ext != prev2)).astype(jnp.float32)
    lpt = jnp.transpose(log_probs, (1, 0, 2))
    tlen = target_lengths.astype(jnp.float32)[:, None]
    T = log_probs.shape[0]

    enc, stats = pl.pallas_call(
        _enc_kernel,
        out_shape=(jax.ShapeDtypeStruct((16, 384, 384), jnp.int32),
                   jax.ShapeDtypeStruct((16, 128), jnp.float32)),
    )(gh_label, gah_label, text_map, link_map, conf_map)

    scres = _sc_select(enc.reshape(16, _PN), stats)

    out = pl.pallas_call(
        _rest_kernel,
        out_shape=jax.ShapeDtypeStruct((8, 128), jnp.float32),
        scratch_shapes=[pltpu.VMEM((T, N, L), jnp.float32)],
    )(stats, scres, a_logits, p_logits, a_label[:, None], p_label[:, None],
      lpt, ext, allow_skip, tlen)
    return out[0, 0], out[0, 1], out[0, 2], out[0, 3]


def kernel(gh_label, gah_label, text_map, link_map, conf_map, a_logits,
           p_logits, a_label, p_label, log_probs, targets, target_lengths):
    return _run(gh_label, gah_label, text_map, link_map, conf_map, a_logits,
                p_logits, a_label, p_label, log_probs, targets, target_lengths)
